# baseline (speedup 1.0000x reference)
import functools

import jax
import jax.numpy as jnp
from jax import lax
from jax.experimental import pallas as pl
from jax.experimental.pallas import tpu as pltpu
from jax.experimental.pallas import tpu_sc as plsc

BATCH = 16384
N_COLS = 1000
K = 256
THRESHOLD = 0.5

NUM_CORES = 2
NUM_SUBCORES = 16
NUM_WORKERS = NUM_CORES * NUM_SUBCORES
COLS_PER_WORKER = BATCH // NUM_WORKERS
W = 128
NUM_SLABS = COLS_PER_WORKER // W
LANES = 16
HALF_K = K // 2


def _sc_kernel(
    sim_hbm, idxlo_hbm, idxhi_hbm, out_hbm,
    idx_lo, idx_hi, slab0, slab1, out_v,
    sem_in0, sem_in1, sem_out,
):
    wid = lax.axis_index("s") * NUM_CORES + lax.axis_index("c")
    col_base = wid * COLS_PER_WORKER
    ones = jnp.full((LANES,), 1.0, jnp.float32)
    zeros = jnp.zeros((LANES,), jnp.float32)
    iota = lax.iota(jnp.int32, LANES)
    perms = [(iota + s) & (LANES - 1) for s in range(LANES)]

    pltpu.sync_copy(idxlo_hbm, idx_lo)
    pltpu.sync_copy(idxhi_hbm, idx_hi)

    slab_bufs = (slab0, slab1)
    sem_in = (sem_in0, sem_in1)

    def start_in(s, b):
        c0 = col_base + s * W
        lo = pltpu.async_copy(
            sim_hbm.at[idx_lo, pl.ds(c0, W)],
            slab_bufs[b].at[pl.ds(0, HALF_K)],
            sem_in[b],
        )
        hi = pltpu.async_copy(
            sim_hbm.at[idx_hi, pl.ds(c0, W)],
            slab_bufs[b].at[pl.ds(HALF_K, HALF_K)],
            sem_in[b],
        )
        return lo, hi

    in_cp = {0: start_in(0, 0), 1: start_in(1, 1)}
    out_cp = None
    for s in range(NUM_SLABS):
        b = s % 2
        lo, hi = in_cp.pop(s)
        lo.wait()
        hi.wait()
        if out_cp is not None:
            out_cp.wait()
        slab_v = slab_bufs[b]

        @plsc.parallel_loop(0, (W // LANES) * (K // LANES), step=1, unroll=6)
        def block_body(q, slab_v=slab_v):
            w0 = (q >> 4) << 4
            j0 = (q & (LANES - 1)) << 4
            jvec = jnp.full((LANES,), j0, jnp.int32) + iota
            w0vec = jnp.full((LANES,), w0, jnp.int32)
            for d in range(LANES):
                wvec = w0vec + perms[d]
                g = plsc.load_gather(slab_v, [jvec, wvec])
                v = jnp.where(g >= THRESHOLD, ones, zeros)
                plsc.store_scatter(out_v, [wvec, jvec], v)

        out_cp = pltpu.async_copy(
            out_v, out_hbm.at[pl.ds(col_base + s * W, W)], sem_out
        )
        if s + 2 < NUM_SLABS:
            in_cp[s + 2] = start_in(s + 2, b)
    out_cp.wait()


_call = functools.partial(
    pl.kernel,
    out_type=jax.ShapeDtypeStruct((BATCH, K), jnp.float32),
    mesh=plsc.VectorSubcoreMesh(core_axis_name="c", subcore_axis_name="s"),
    compiler_params=pltpu.CompilerParams(
        use_tc_tiling_on_sc=True, needs_layout_passes=False
    ),
    scratch_types=[
        pltpu.VMEM((HALF_K,), jnp.int32),
        pltpu.VMEM((HALF_K,), jnp.int32),
        pltpu.VMEM((K, W), jnp.float32),
        pltpu.VMEM((K, W), jnp.float32),
        pltpu.VMEM((W, K), jnp.float32),
        pltpu.SemaphoreType.DMA,
        pltpu.SemaphoreType.DMA,
        pltpu.SemaphoreType.DMA,
    ],
)(_sc_kernel)


def kernel(similarities, label_indices):
    return _call(
        similarities.T,
        label_indices[:HALF_K],
        label_indices[HALF_K:],
    )


# speedup vs baseline: 1.3007x; 1.3007x over previous
import functools

import jax
import jax.numpy as jnp
from jax import lax
from jax.experimental import pallas as pl
from jax.experimental.pallas import tpu as pltpu
from jax.experimental.pallas import tpu_sc as plsc

BATCH = 16384
N_COLS = 1000
K = 256
THRESHOLD = 0.5

NUM_CORES = 2
NUM_SUBCORES = 16
NUM_WORKERS = NUM_CORES * NUM_SUBCORES
COLS_PER_WORKER = BATCH // NUM_WORKERS
W = 128
NUM_SLABS = COLS_PER_WORKER // W
LANES = 16
HALF_K = K // 2


def _sc_kernel(
    sim_hbm, idx_hbm, out_hbm,
    idx_v, slab0, slab1, out_v,
    sem_in0, sem_in1, sem_out,
):
    wid = lax.axis_index("s") * NUM_CORES + lax.axis_index("c")
    col_base = wid * COLS_PER_WORKER
    ones = jnp.full((LANES,), 1.0, jnp.float32)
    zeros = jnp.zeros((LANES,), jnp.float32)
    iota = lax.iota(jnp.int32, LANES)
    perms = [(iota + s) & (LANES - 1) for s in range(LANES)]

    pltpu.sync_copy(idx_hbm, idx_v)
    idx_lo = idx_v.at[pl.ds(0, HALF_K)]
    idx_hi = idx_v.at[pl.ds(HALF_K, HALF_K)]

    slab_bufs = (slab0, slab1)
    sem_in = (sem_in0, sem_in1)

    def start_in(s, b):
        c0 = col_base + s * W
        lo = pltpu.async_copy(
            sim_hbm.at[idx_lo, pl.ds(c0, W)],
            slab_bufs[b].at[pl.ds(0, HALF_K)],
            sem_in[b],
        )
        hi = pltpu.async_copy(
            sim_hbm.at[idx_hi, pl.ds(c0, W)],
            slab_bufs[b].at[pl.ds(HALF_K, HALF_K)],
            sem_in[b],
        )
        return lo, hi

    in_cp = {0: start_in(0, 0), 1: start_in(1, 1)}
    out_cp = None
    for s in range(NUM_SLABS):
        b = s % 2
        lo, hi = in_cp.pop(s)
        lo.wait()
        hi.wait()
        if out_cp is not None:
            out_cp.wait()
        slab_v = slab_bufs[b]

        @plsc.parallel_loop(0, (W // LANES) * (K // LANES), step=1, unroll=4)
        def block_body(q, slab_v=slab_v):
            w0 = (q >> 4) << 4
            j0 = (q & (LANES - 1)) << 4
            jvec = jnp.full((LANES,), j0, jnp.int32) + iota
            w0vec = jnp.full((LANES,), w0, jnp.int32)
            for d in range(LANES):
                wvec = w0vec + perms[d]
                g = plsc.load_gather(slab_v, [jvec, wvec])
                v = jnp.where(g >= THRESHOLD, ones, zeros)
                plsc.store_scatter(out_v, [wvec, jvec], v)

        out_cp = pltpu.async_copy(
            out_v, out_hbm.at[pl.ds(col_base + s * W, W)], sem_out
        )
        if s + 2 < NUM_SLABS:
            in_cp[s + 2] = start_in(s + 2, b)
    out_cp.wait()


_call = functools.partial(
    pl.kernel,
    out_type=jax.ShapeDtypeStruct((BATCH, K), jnp.float32),
    mesh=plsc.VectorSubcoreMesh(core_axis_name="c", subcore_axis_name="s"),
    compiler_params=pltpu.CompilerParams(
        use_tc_tiling_on_sc=True, needs_layout_passes=False
    ),
    scratch_types=[
        pltpu.VMEM((K,), jnp.int32),
        pltpu.VMEM((K, W), jnp.float32),
        pltpu.VMEM((K, W), jnp.float32),
        pltpu.VMEM((W, K), jnp.float32),
        pltpu.SemaphoreType.DMA,
        pltpu.SemaphoreType.DMA,
        pltpu.SemaphoreType.DMA,
    ],
)(_sc_kernel)


def kernel(similarities, label_indices):
    return _call(similarities.T, label_indices)
